# baseline (speedup 1.0000x reference)
import jax
import jax.numpy as jnp
from jax import lax
from jax.experimental import pallas as pl
from jax.experimental.pallas import tpu as pltpu
from jax.experimental.pallas import tpu_sc as plsc

N_PTS = 131072
N_LVL = 16
FEAT = 4
TABLE = 524288
MASK = TABLE - 1
PRIME = -1640531535
RES_LIST = [int(16 * 1.5 ** i) for i in range(N_LVL)]

NC, NS = 2, 16
NW = NC * NS
PPW = N_PTS // NW
CHUNK = 128
NCH = PPW // CHUNK
GRP = CHUNK // 16

GROUPS = N_LVL * TABLE * FEAT // 16
GPW = GROUPS // NS
RCH = 1024
RIT = GPW // RCH


def _encode_body(x_hbm, tab_hbm, out_hbm, scr_hbm,
                 x_v, idx_v0, idx_v1, su4_v0, su4_v1, rows_v0, rows_v1,
                 wx_v0, wx_v1, wy_v0, wy_v1, out_v,
                 semg0, semg1, semi0, semi1, semo0, semo1):
    i32 = jnp.int32
    core = lax.axis_index("c")
    sid = lax.axis_index("s")
    wid = sid * i32(NC) + core
    sc_off = core * i32(GROUPS)

    iota = lax.iota(jnp.int32, 16)
    zeros_i = jnp.zeros((16,), jnp.int32)
    ones_i = jnp.ones((16,), jnp.int32)
    io_div4 = iota >> 2
    io_mod4x4 = (iota & i32(3)) * i32(4)

    rows_b = (rows_v0, rows_v1)
    semi_b = (semi0, semi1)
    semo_b = (semo0, semo1)

    g_base = sid * i32(GPW)

    pltpu.async_copy(tab_hbm.at[pl.ds(g_base, RCH)], rows_v0.at[pl.ds(0, RCH)], semi0)
    pltpu.async_copy(tab_hbm.at[pl.ds(g_base + i32(RCH), RCH)],
                     rows_v1.at[pl.ds(0, RCH)], semi1)

    def ring_body(k, _):
        for par in range(2):
            cb = k * i32(2) + i32(par)
            rv, si, so = rows_b[par], semi_b[par], semo_b[par]
            pltpu.make_async_copy(tab_hbm.at[pl.ds(0, RCH)],
                                  rv.at[pl.ds(0, RCH)], si).wait()
            @pl.when(cb >= i32(2))
            def _():
                pltpu.make_async_copy(tab_hbm.at[pl.ds(0, RCH)],
                                      rv.at[pl.ds(RCH, RCH)], so).wait()

            def block_body(bb, _):
                b32 = bb * i32(32)
                for f in range(FEAT):
                    for k8 in range(8):
                        v = rv[b32 + i32(f * 8 + k8), :]
                        plsc.store_scatter(
                            rv,
                            [i32(RCH) + b32 + i32(k8 * 4) + io_div4,
                             io_mod4x4 + i32(f)],
                            v)
                return ()

            lax.fori_loop(jnp.int32(0), jnp.int32(RCH // 32), block_body, (),
                          unroll=False)
            og = sc_off + g_base + cb * i32(RCH)
            pltpu.async_copy(rv.at[pl.ds(RCH, RCH)],
                             scr_hbm.at[pl.ds(og, RCH)], so)

            @pl.when(cb + i32(2) < i32(RIT))
            def _():
                pltpu.async_copy(
                    tab_hbm.at[pl.ds(g_base + (cb + i32(2)) * i32(RCH), RCH)],
                    rv.at[pl.ds(0, RCH)], si)
        return ()

    lax.fori_loop(jnp.int32(0), jnp.int32(RIT // 2), ring_body, (), unroll=False)
    for par in range(2):
        pltpu.make_async_copy(tab_hbm.at[pl.ds(0, RCH)],
                              rows_b[par].at[pl.ds(RCH, RCH)], semo_b[par]).wait()
    plsc.subcore_barrier()

    idx_b = (idx_v0, idx_v1)
    su4_b = (su4_v0, su4_v1)
    wx_b = (wx_v0, wx_v1)
    wy_b = (wy_v0, wy_v1)
    semg_b = (semg0, semg1)

    def chunk_body(ci, _):
        base = wid * i32(PPW) + ci * i32(CHUNK)
        pltpu.sync_copy(x_hbm.at[pl.ds(base, CHUNK)], x_v)

        def hash_level(l, bi):
            res = RES_LIST[l] * 1.0
            lvl_off = sc_off + i32(l * (TABLE // 4))
            idx_v, su4_v, wx_v, wy_v = idx_b[bi], su4_b[bi], wx_b[bi], wy_b[bi]

            def hash_body(g, _):
                g16 = g * i32(16)
                pidx = g16 + iota
                xx = plsc.load_gather(x_v, [pidx, zeros_i])
                yy = plsc.load_gather(x_v, [pidx, ones_i])
                px = (xx + 1.0) * 0.5 * res
                py = (yy + 1.0) * 0.5 * res
                fx = px.astype(jnp.int32)
                fy = py.astype(jnp.int32)
                wx_v[pl.ds(g16, 16)] = px - fx.astype(jnp.float32)
                wy_v[pl.ds(g16, 16)] = py - fy.astype(jnp.float32)
                hb = fx + fy * i32(PRIME)
                hs = (hb & i32(MASK),
                      (hb + i32(PRIME)) & i32(MASK),
                      (hb + i32(1)) & i32(MASK),
                      (hb + i32(PRIME + 1)) & i32(MASK))
                for c in range(4):
                    h = hs[c]
                    idx_v[pl.ds(g16 + i32(c * CHUNK), 16)] = (h >> 2) + lvl_off
                    su4_v[pl.ds(g16 + i32(c * CHUNK), 16)] = (h & i32(3)) * i32(4)
                return ()

            lax.fori_loop(jnp.int32(0), jnp.int32(GRP), hash_body, (), unroll=False)
            return pltpu.async_copy(scr_hbm.at[idx_v],
                                    rows_b[bi].at[pl.ds(0, 4 * CHUNK)], semg_b[bi])

        def comb_level(l, bi):
            su4_v, rows_v = su4_b[bi], rows_b[bi]
            wx_v, wy_v = wx_b[bi], wy_b[bi]

            def comb_body(g, _):
                g16 = g * i32(16)
                pidx = g16 + iota
                wx = wx_v[pl.ds(g16, 16)]
                wy = wy_v[pl.ds(g16, 16)]
                ws = ((1.0 - wx) * (1.0 - wy), (1.0 - wx) * wy,
                      wx * (1.0 - wy), wx * wy)
                ss = tuple(su4_v[pl.ds(g16 + i32(c * CHUNK), 16)] for c in range(4))
                for f in range(FEAT):
                    acc = None
                    for c in range(4):
                        v = plsc.load_gather(
                            rows_v, [pidx + i32(c * CHUNK), ss[c] + i32(f)])
                        acc = ws[c] * v if acc is None else acc + ws[c] * v
                    plsc.store_scatter(out_v, [pidx, zeros_i + i32(4 * l + f)], acc)
                return ()

            lax.fori_loop(jnp.int32(0), jnp.int32(GRP), comb_body, (), unroll=False)

        cp = hash_level(0, 0)
        for l in range(N_LVL):
            cp_next = hash_level(l + 1, (l + 1) % 2) if l + 1 < N_LVL else None
            cp.wait()
            comb_level(l, l % 2)
            cp = cp_next

        pltpu.sync_copy(out_v, out_hbm.at[pl.ds(base, CHUNK)])
        return ()

    lax.fori_loop(jnp.int32(0), jnp.int32(NCH), chunk_body, (), unroll=False)


@jax.jit
def _encode(x, hash_latents):
    mesh = plsc.VectorSubcoreMesh(core_axis_name="c", subcore_axis_name="s")
    tab16 = (hash_latents.reshape(N_LVL * TABLE // 128, 128, FEAT)
             .transpose(0, 2, 1)
             .reshape(GROUPS, 16))
    out, _ = pl.kernel(
        _encode_body,
        out_type=[
            jax.ShapeDtypeStruct((N_PTS, N_LVL * FEAT), jnp.float32),
            jax.ShapeDtypeStruct((NC * GROUPS, 16), jnp.float32),
        ],
        mesh=mesh,
        compiler_params=pltpu.CompilerParams(
            needs_layout_passes=False, use_tc_tiling_on_sc=False),
        scratch_types=[
            pltpu.VMEM((CHUNK, 2), jnp.float32),
            pltpu.VMEM((4 * CHUNK,), jnp.int32),
            pltpu.VMEM((4 * CHUNK,), jnp.int32),
            pltpu.VMEM((4 * CHUNK,), jnp.int32),
            pltpu.VMEM((4 * CHUNK,), jnp.int32),
            pltpu.VMEM((2 * RCH, 16), jnp.float32),
            pltpu.VMEM((2 * RCH, 16), jnp.float32),
            pltpu.VMEM((CHUNK,), jnp.float32),
            pltpu.VMEM((CHUNK,), jnp.float32),
            pltpu.VMEM((CHUNK,), jnp.float32),
            pltpu.VMEM((CHUNK,), jnp.float32),
            pltpu.VMEM((CHUNK, N_LVL * FEAT), jnp.float32),
            pltpu.SemaphoreType.DMA,
            pltpu.SemaphoreType.DMA,
            pltpu.SemaphoreType.DMA,
            pltpu.SemaphoreType.DMA,
            pltpu.SemaphoreType.DMA,
            pltpu.SemaphoreType.DMA,
        ],
    )(x, tab16)
    return out


def kernel(x, hash_latents):
    return _encode(x, hash_latents)
